# baseline (speedup 1.0000x reference)
import jax
import jax.numpy as jnp
from jax import lax
from jax.experimental import pallas as pl
from jax.experimental.pallas import tpu as pltpu
from jax.experimental.pallas import tpu_sc as plsc

N = 10000
E = 320000
D_IN = 128
D_HID = 256
NP = 10240
NC = 2
NS = 16
NW = NC * NS
RPW = NP // NW
B_E = 4000
CAP = 11520
K = 32
ACC_ROWS = RPW + 1
DUMP = RPW
RB = 2560
GRID = NP // RB


def _sc_mesh():
    return plsc.VectorSubcoreMesh(
        core_axis_name="c", subcore_axis_name="s", num_cores=NC, num_subcores=NS
    )


_SC_PARAMS = pltpu.CompilerParams(needs_layout_passes=False)


def _part_body(src_hbm, dst_hbm, fsrc_hbm, fdst_hbm, cnt_hbm, deg_hbm,
               src_v0, dst_v0, src_v1, dst_v1, fsrc, fdst, deg_v, cnt_v,
               sem_s0, sem_d0, sem_s1, sem_d1):
    c = lax.axis_index("c")
    s = lax.axis_index("s")
    w = s * NC + c
    lo = w * RPW

    def zdeg(i, _):
        deg_v[pl.ds(i * 16, 16)] = jnp.zeros((16,), jnp.float32)
        return 0

    lax.fori_loop(0, RPW // 16, zdeg, 0)

    z16 = jnp.zeros((16,), jnp.int32)
    d16 = jnp.full((16,), DUMP, jnp.int32)

    def pre(i, _):
        fsrc[pl.ds(i * 16, 16)] = z16
        fdst[pl.ds(i * 16, 16)] = d16
        return 0

    lax.fori_loop(0, CAP // 16, pre, 0)

    ones = jnp.ones((16,), jnp.float32)
    NBLK = E // B_E

    def fetch(src_v, dst_v, sem_s, sem_d, blk):
        base_e = jnp.minimum(blk, NBLK - 1) * B_E
        pltpu.async_copy(src_hbm.at[pl.ds(base_e, B_E)], src_v, sem_s)
        pltpu.async_copy(dst_hbm.at[pl.ds(base_e, B_E)], dst_v, sem_d)

    def wait(src_v, dst_v, sem_s, sem_d):
        pltpu.make_async_copy(src_hbm.at[pl.ds(0, B_E)], src_v, sem_s).wait()
        pltpu.make_async_copy(dst_hbm.at[pl.ds(0, B_E)], dst_v, sem_d).wait()

    def filter_block(src_v, dst_v, pos):
        def filt(i, pos):
            for h in range(2):
                sl = src_v[pl.ds(i * 32 + h * 16, 16)]
                dl = dst_v[pl.ds(i * 32 + h * 16, 16)] - lo
                m = (dl >= 0) & (dl < RPW)
                dl_safe = jnp.clip(dl, 0, RPW - 1)
                pos_c = jnp.minimum(pos, CAP - 16)
                plsc.store_compressed(fsrc.at[pl.ds(pos_c, 16)], sl, mask=m)
                plsc.store_compressed(fdst.at[pl.ds(pos_c, 16)], dl_safe, mask=m)
                plsc.addupdate_scatter(deg_v, [dl_safe], ones, mask=m)
                pos = pos + jnp.sum(m.astype(jnp.int32))
            return pos

        return lax.fori_loop(0, B_E // 32, filt, pos)

    fetch(src_v0, dst_v0, sem_s0, sem_d0, 0)

    def pair(p, pos):
        fetch(src_v1, dst_v1, sem_s1, sem_d1, 2 * p + 1)
        wait(src_v0, dst_v0, sem_s0, sem_d0)
        pos = filter_block(src_v0, dst_v0, pos)
        fetch(src_v0, dst_v0, sem_s0, sem_d0, 2 * p + 2)
        wait(src_v1, dst_v1, sem_s1, sem_d1)
        return filter_block(src_v1, dst_v1, pos)

    pos = lax.fori_loop(0, NBLK // 2, pair, jnp.int32(0))
    wait(src_v0, dst_v0, sem_s0, sem_d0)

    cnt_v[pl.ds(0, 16)] = jnp.broadcast_to(jnp.minimum(pos, CAP), (16,))
    pltpu.sync_copy(fsrc, fsrc_hbm.at[w])
    pltpu.sync_copy(fdst, fdst_hbm.at[w])
    pltpu.sync_copy(cnt_v, cnt_hbm.at[w])
    pltpu.sync_copy(deg_v, deg_hbm.at[w])


def _sc_partition(src32, dst32):
    return pl.kernel(
        _part_body,
        out_type=[
            jax.ShapeDtypeStruct((NW, CAP), jnp.int32),
            jax.ShapeDtypeStruct((NW, CAP), jnp.int32),
            jax.ShapeDtypeStruct((NW, 16), jnp.int32),
            jax.ShapeDtypeStruct((NW, RPW), jnp.float32),
        ],
        mesh=_sc_mesh(),
        compiler_params=_SC_PARAMS,
        scratch_types=[
            pltpu.VMEM((B_E,), jnp.int32),
            pltpu.VMEM((B_E,), jnp.int32),
            pltpu.VMEM((B_E,), jnp.int32),
            pltpu.VMEM((B_E,), jnp.int32),
            pltpu.VMEM((CAP,), jnp.int32),
            pltpu.VMEM((CAP,), jnp.int32),
            pltpu.VMEM((RPW,), jnp.float32),
            pltpu.VMEM((16,), jnp.int32),
            pltpu.SemaphoreType.DMA,
            pltpu.SemaphoreType.DMA,
            pltpu.SemaphoreType.DMA,
            pltpu.SemaphoreType.DMA,
        ],
    )(src32, dst32)


CAPX = CAP + K


def _segsum_body(u_hbm, fsrc_hbm, fdst_hbm, cnt_hbm, out_hbm,
                 fsrc, fdst, rows0, rows1, cnt_v, acc,
                 sem0, sem1):
    c = lax.axis_index("c")
    s = lax.axis_index("s")
    w = s * NC + c

    def zacc(i, _):
        acc[i // 16, pl.ds((i % 16) * 16, 16)] = jnp.zeros((16,), jnp.float32)
        return 0

    lax.fori_loop(0, ACC_ROWS * 16, zacc, 0)

    pltpu.sync_copy(fsrc_hbm.at[w], fsrc.at[pl.ds(0, CAP)])
    pltpu.sync_copy(fdst_hbm.at[w], fdst.at[pl.ds(0, CAP)])
    pltpu.sync_copy(cnt_hbm.at[w], cnt_v)
    z16 = jnp.zeros((16,), jnp.int32)
    d16 = jnp.full((16,), DUMP, jnp.int32)
    for j in range(K // 16):
        fsrc[pl.ds(CAP + j * 16, 16)] = z16
        fdst[pl.ds(CAP + j * 16, 16)] = d16

    count = cnt_v[pl.ds(0, 16)][0]
    pairs = (count + (2 * K - 1)) // (2 * K)

    def add_batch(rows, t):
        base = t * K

        def per_group(g, _):
            dv = fdst[pl.ds(base + g * 16, 16)]
            for l in range(16):
                dloc = dv[l]
                i = g * 16 + l
                for j in range(D_HID // 16):
                    acc[dloc, pl.ds(j * 16, 16)] = (
                        acc[dloc, pl.ds(j * 16, 16)]
                        + rows[i, pl.ds(j * 16, 16)])
            return 0

        lax.fori_loop(0, K // 16, per_group, 0)

    def gather_at(t):
        return u_hbm.at[fsrc.at[pl.ds(t * K, K)]]

    pltpu.async_copy(gather_at(0), rows0, sem0)

    def pair(p, _):
        t0 = 2 * p
        pltpu.async_copy(gather_at(t0 + 1), rows1, sem1)
        pltpu.make_async_copy(gather_at(t0), rows0, sem0).wait()
        add_batch(rows0, t0)
        pltpu.async_copy(gather_at(t0 + 2), rows0, sem0)
        pltpu.make_async_copy(gather_at(t0 + 1), rows1, sem1).wait()
        add_batch(rows1, t0 + 1)
        return 0

    lax.fori_loop(0, pairs, pair, 0)
    pltpu.make_async_copy(gather_at(0), rows0, sem0).wait()

    pltpu.sync_copy(acc.at[pl.ds(0, RPW)], out_hbm.at[pl.ds(w * RPW, RPW)])


def _sc_segsum(u, fsrc, fdst, cnt):
    return pl.kernel(
        _segsum_body,
        out_type=jax.ShapeDtypeStruct((NP, D_HID), jnp.float32),
        mesh=_sc_mesh(),
        compiler_params=_SC_PARAMS,
        scratch_types=[
            pltpu.VMEM((CAPX,), jnp.int32),
            pltpu.VMEM((CAPX,), jnp.int32),
            pltpu.VMEM((K, D_HID), jnp.float32),
            pltpu.VMEM((K, D_HID), jnp.float32),
            pltpu.VMEM((16,), jnp.int32),
            pltpu.VMEM((ACC_ROWS, D_HID), jnp.float32),
            pltpu.SemaphoreType.DMA,
            pltpu.SemaphoreType.DMA,
        ],
    )(u, fsrc, fdst, cnt)


def _proj1_body(x_r, w_r, dg_r, u1_r, dv_r):
    deg = dg_r[0, :] + 1.0
    dinv = lax.rsqrt(jnp.maximum(deg, 1e-12))
    dv_r[...] = dinv[None, :]
    u1_r[...] = jnp.dot(x_r[...] * dinv[:, None], w_r[...],
                        preferred_element_type=jnp.float32)


def _tc_proj1(x_pad, W1, deg):
    return pl.pallas_call(
        _proj1_body,
        grid=(GRID,),
        in_specs=[
            pl.BlockSpec((RB, D_IN), lambda i: (i, 0)),
            pl.BlockSpec((D_IN, D_HID), lambda i: (0, 0)),
            pl.BlockSpec((1, RB), lambda i: (0, i)),
        ],
        out_specs=[
            pl.BlockSpec((RB, D_HID), lambda i: (i, 0)),
            pl.BlockSpec((1, RB), lambda i: (0, i)),
        ],
        out_shape=[
            jax.ShapeDtypeStruct((NP, D_HID), jnp.float32),
            jax.ShapeDtypeStruct((1, NP), jnp.float32),
        ],
    )(x_pad, W1, deg)


def _ln(v, g, b, eps=1e-5):
    mean = jnp.mean(v, axis=-1, keepdims=True)
    var = jnp.mean(jnp.square(v - mean), axis=-1, keepdims=True)
    return (v - mean) * lax.rsqrt(var + eps) * g + b


def _mid_body(s1_r, u1_r, dv_r, b1_r, g1_r, be1_r, g2_r, be2_r, w2_r,
              h_r, u2_r):
    dinv = dv_r[0, :]
    conv1 = dinv[:, None] * (s1_r[...] + u1_r[...]) + b1_r[...]
    h = jax.nn.relu(_ln(conv1, g1_r[...], be1_r[...]))
    t = jax.nn.relu(_ln(h, g2_r[...], be2_r[...]))
    h_r[...] = h
    u2_r[...] = jnp.dot(t * dinv[:, None], w2_r[...],
                        preferred_element_type=jnp.float32)


def _tc_mid(s1, u1, dinv, b1, g1, be1, g2, be2, W2):
    vec = lambda: pl.BlockSpec((1, D_HID), lambda i: (0, 0))
    return pl.pallas_call(
        _mid_body,
        grid=(GRID,),
        in_specs=[
            pl.BlockSpec((RB, D_HID), lambda i: (i, 0)),
            pl.BlockSpec((RB, D_HID), lambda i: (i, 0)),
            pl.BlockSpec((1, RB), lambda i: (0, i)),
            vec(), vec(), vec(), vec(), vec(),
            pl.BlockSpec((D_HID, D_HID), lambda i: (0, 0)),
        ],
        out_specs=[
            pl.BlockSpec((RB, D_HID), lambda i: (i, 0)),
            pl.BlockSpec((RB, D_HID), lambda i: (i, 0)),
        ],
        out_shape=[
            jax.ShapeDtypeStruct((NP, D_HID), jnp.float32),
            jax.ShapeDtypeStruct((NP, D_HID), jnp.float32),
        ],
    )(s1, u1, dinv, b1, g1, be1, g2, be2, W2)


def _fin_body(h_r, dv_r, s2_r, u2_r, b2_r, o_r):
    dinv = dv_r[0, :]
    o_r[...] = h_r[...] + dinv[:, None] * (s2_r[...] + u2_r[...]) + b2_r[...]


def _tc_fin(h, dinv, s2, u2, b2):
    return pl.pallas_call(
        _fin_body,
        grid=(GRID,),
        in_specs=[
            pl.BlockSpec((RB, D_HID), lambda i: (i, 0)),
            pl.BlockSpec((1, RB), lambda i: (0, i)),
            pl.BlockSpec((RB, D_HID), lambda i: (i, 0)),
            pl.BlockSpec((RB, D_HID), lambda i: (i, 0)),
            pl.BlockSpec((1, D_HID), lambda i: (0, 0)),
        ],
        out_specs=pl.BlockSpec((RB, D_HID), lambda i: (i, 0)),
        out_shape=jax.ShapeDtypeStruct((NP, D_HID), jnp.float32),
    )(h, dinv, s2, u2, b2)


def kernel(x, edge_index, W1, b1, g1, be1, W2, b2, g2, be2):
    src32 = edge_index[0].astype(jnp.int32)
    dst32 = edge_index[1].astype(jnp.int32)
    x_pad = jnp.pad(x, ((0, NP - N), (0, 0)))

    fsrc, fdst, cnt, deg = _sc_partition(src32, dst32)
    deg = deg.reshape(1, NP)
    u1, dinv = _tc_proj1(x_pad, W1, deg)
    s1 = _sc_segsum(u1, fsrc, fdst, cnt)
    h, u2 = _tc_mid(s1, u1, dinv,
                    b1.reshape(1, -1), g1.reshape(1, -1), be1.reshape(1, -1),
                    g2.reshape(1, -1), be2.reshape(1, -1), W2)
    s2 = _sc_segsum(u2, fsrc, fdst, cnt)
    out = _tc_fin(h, dinv, s2, u2, b2.reshape(1, -1))
    return out[:N]


# speedup vs baseline: 1.2918x; 1.2918x over previous
import jax
import jax.numpy as jnp
from jax import lax
from jax.experimental import pallas as pl
from jax.experimental.pallas import tpu as pltpu
from jax.experimental.pallas import tpu_sc as plsc

N = 10000
E = 320000
D_IN = 128
D_HID = 256
NP = 10240
NC = 2
NS = 16
NW = NC * NS
RPW = NP // NW
B_E = 4000
CAP = 11520
K = 32
ACC_ROWS = RPW + 1
DUMP = RPW
RB = 2560
GRID = NP // RB


def _sc_mesh():
    return plsc.VectorSubcoreMesh(
        core_axis_name="c", subcore_axis_name="s", num_cores=NC, num_subcores=NS
    )


_SC_PARAMS = pltpu.CompilerParams(needs_layout_passes=False)


def _part_body(src_hbm, dst_hbm, fsrc_hbm, fdst_hbm, cnt_hbm, deg_hbm,
               src_v0, dst_v0, src_v1, dst_v1, fsrc, fdst, deg_v, cnt_v,
               sem_s0, sem_d0, sem_s1, sem_d1):
    c = lax.axis_index("c")
    s = lax.axis_index("s")
    w = s * NC + c
    lo = w * RPW

    def zdeg(i, _):
        deg_v[pl.ds(i * 16, 16)] = jnp.zeros((16,), jnp.float32)
        return 0

    lax.fori_loop(0, RPW // 16, zdeg, 0)

    z16 = jnp.zeros((16,), jnp.int32)
    d16 = jnp.full((16,), DUMP, jnp.int32)

    def pre(i, _):
        fsrc[pl.ds(i * 16, 16)] = z16
        fdst[pl.ds(i * 16, 16)] = d16
        return 0

    lax.fori_loop(0, CAP // 16, pre, 0)

    ones = jnp.ones((16,), jnp.float32)
    NBLK = E // B_E

    def fetch(src_v, dst_v, sem_s, sem_d, blk):
        base_e = jnp.minimum(blk, NBLK - 1) * B_E
        pltpu.async_copy(src_hbm.at[pl.ds(base_e, B_E)], src_v, sem_s)
        pltpu.async_copy(dst_hbm.at[pl.ds(base_e, B_E)], dst_v, sem_d)

    def wait(src_v, dst_v, sem_s, sem_d):
        pltpu.make_async_copy(src_hbm.at[pl.ds(0, B_E)], src_v, sem_s).wait()
        pltpu.make_async_copy(dst_hbm.at[pl.ds(0, B_E)], dst_v, sem_d).wait()

    def filter_block(src_v, dst_v, pos):
        def filt(i, pos):
            for h in range(2):
                sl = src_v[pl.ds(i * 32 + h * 16, 16)]
                dl = dst_v[pl.ds(i * 32 + h * 16, 16)] - lo
                m = (dl >= 0) & (dl < RPW)
                dl_safe = jnp.clip(dl, 0, RPW - 1)
                pos_c = jnp.minimum(pos, CAP - 16)
                plsc.store_compressed(fsrc.at[pl.ds(pos_c, 16)], sl, mask=m)
                plsc.store_compressed(fdst.at[pl.ds(pos_c, 16)], dl_safe, mask=m)
                plsc.addupdate_scatter(deg_v, [dl_safe], ones, mask=m)
                pos = pos + jnp.sum(m.astype(jnp.int32))
            return pos

        return lax.fori_loop(0, B_E // 32, filt, pos)

    fetch(src_v0, dst_v0, sem_s0, sem_d0, 0)

    def pair(p, pos):
        fetch(src_v1, dst_v1, sem_s1, sem_d1, 2 * p + 1)
        wait(src_v0, dst_v0, sem_s0, sem_d0)
        pos = filter_block(src_v0, dst_v0, pos)
        fetch(src_v0, dst_v0, sem_s0, sem_d0, 2 * p + 2)
        wait(src_v1, dst_v1, sem_s1, sem_d1)
        return filter_block(src_v1, dst_v1, pos)

    pos = lax.fori_loop(0, NBLK // 2, pair, jnp.int32(0))
    wait(src_v0, dst_v0, sem_s0, sem_d0)

    cnt_v[pl.ds(0, 16)] = jnp.broadcast_to(jnp.minimum(pos, CAP), (16,))
    pltpu.sync_copy(fsrc, fsrc_hbm.at[w])
    pltpu.sync_copy(fdst, fdst_hbm.at[w])
    pltpu.sync_copy(cnt_v, cnt_hbm.at[w])
    pltpu.sync_copy(deg_v, deg_hbm.at[w])


def _sc_partition(src32, dst32):
    return pl.kernel(
        _part_body,
        out_type=[
            jax.ShapeDtypeStruct((NW, CAP), jnp.int32),
            jax.ShapeDtypeStruct((NW, CAP), jnp.int32),
            jax.ShapeDtypeStruct((NW, 16), jnp.int32),
            jax.ShapeDtypeStruct((NW, RPW), jnp.float32),
        ],
        mesh=_sc_mesh(),
        compiler_params=_SC_PARAMS,
        scratch_types=[
            pltpu.VMEM((B_E,), jnp.int32),
            pltpu.VMEM((B_E,), jnp.int32),
            pltpu.VMEM((B_E,), jnp.int32),
            pltpu.VMEM((B_E,), jnp.int32),
            pltpu.VMEM((CAP,), jnp.int32),
            pltpu.VMEM((CAP,), jnp.int32),
            pltpu.VMEM((RPW,), jnp.float32),
            pltpu.VMEM((16,), jnp.int32),
            pltpu.SemaphoreType.DMA,
            pltpu.SemaphoreType.DMA,
            pltpu.SemaphoreType.DMA,
            pltpu.SemaphoreType.DMA,
        ],
    )(src32, dst32)


CAPX = CAP + K


def _segsum_body(u_hbm, fsrc_hbm, fdst_hbm, cnt_hbm, out_hbm,
                 fsrc, fdst, rows0, rows1, cnt_v, acc,
                 sem0, sem1):
    c = lax.axis_index("c")
    s = lax.axis_index("s")
    w = s * NC + c

    def zacc(i, _):
        acc[i // 16, pl.ds((i % 16) * 16, 16)] = jnp.zeros((16,), jnp.float32)
        return 0

    lax.fori_loop(0, ACC_ROWS * 16, zacc, 0)

    pltpu.sync_copy(fsrc_hbm.at[w], fsrc.at[pl.ds(0, CAP)])
    pltpu.sync_copy(fdst_hbm.at[w], fdst.at[pl.ds(0, CAP)])
    pltpu.sync_copy(cnt_hbm.at[w], cnt_v)
    z16 = jnp.zeros((16,), jnp.int32)
    d16 = jnp.full((16,), DUMP, jnp.int32)
    for j in range(K // 16):
        fsrc[pl.ds(CAP + j * 16, 16)] = z16
        fdst[pl.ds(CAP + j * 16, 16)] = d16

    count = cnt_v[pl.ds(0, 16)][0]
    pairs = (count + (2 * K - 1)) // (2 * K)

    def add_batch(rows, t):
        base = t * K

        def per_group(g, _):
            dv = fdst[pl.ds(base + g * 16, 16)]
            for l in range(16):
                dloc = dv[l]
                i = g * 16 + l
                for j in range(D_HID // 16):
                    plsc.addupdate(acc.at[dloc, pl.ds(j * 16, 16)],
                                   rows[i, pl.ds(j * 16, 16)])
            return 0

        lax.fori_loop(0, K // 16, per_group, 0)

    def gather_at(t):
        return u_hbm.at[fsrc.at[pl.ds(t * K, K)]]

    pltpu.async_copy(gather_at(0), rows0, sem0)

    def pair(p, _):
        t0 = 2 * p
        pltpu.async_copy(gather_at(t0 + 1), rows1, sem1)
        pltpu.make_async_copy(gather_at(t0), rows0, sem0).wait()
        add_batch(rows0, t0)
        pltpu.async_copy(gather_at(t0 + 2), rows0, sem0)
        pltpu.make_async_copy(gather_at(t0 + 1), rows1, sem1).wait()
        add_batch(rows1, t0 + 1)
        return 0

    lax.fori_loop(0, pairs, pair, 0)
    pltpu.make_async_copy(gather_at(0), rows0, sem0).wait()

    pltpu.sync_copy(acc.at[pl.ds(0, RPW)], out_hbm.at[pl.ds(w * RPW, RPW)])


def _sc_segsum(u, fsrc, fdst, cnt):
    return pl.kernel(
        _segsum_body,
        out_type=jax.ShapeDtypeStruct((NP, D_HID), jnp.float32),
        mesh=_sc_mesh(),
        compiler_params=_SC_PARAMS,
        scratch_types=[
            pltpu.VMEM((CAPX,), jnp.int32),
            pltpu.VMEM((CAPX,), jnp.int32),
            pltpu.VMEM((K, D_HID), jnp.float32),
            pltpu.VMEM((K, D_HID), jnp.float32),
            pltpu.VMEM((16,), jnp.int32),
            pltpu.VMEM((ACC_ROWS, D_HID), jnp.float32),
            pltpu.SemaphoreType.DMA,
            pltpu.SemaphoreType.DMA,
        ],
    )(u, fsrc, fdst, cnt)


def _proj1_body(x_r, w_r, dg_r, u1_r, dv_r):
    deg = dg_r[0, :] + 1.0
    dinv = lax.rsqrt(jnp.maximum(deg, 1e-12))
    dv_r[...] = dinv[None, :]
    u1_r[...] = jnp.dot(x_r[...] * dinv[:, None], w_r[...],
                        preferred_element_type=jnp.float32)


def _tc_proj1(x_pad, W1, deg):
    return pl.pallas_call(
        _proj1_body,
        grid=(GRID,),
        in_specs=[
            pl.BlockSpec((RB, D_IN), lambda i: (i, 0)),
            pl.BlockSpec((D_IN, D_HID), lambda i: (0, 0)),
            pl.BlockSpec((1, RB), lambda i: (0, i)),
        ],
        out_specs=[
            pl.BlockSpec((RB, D_HID), lambda i: (i, 0)),
            pl.BlockSpec((1, RB), lambda i: (0, i)),
        ],
        out_shape=[
            jax.ShapeDtypeStruct((NP, D_HID), jnp.float32),
            jax.ShapeDtypeStruct((1, NP), jnp.float32),
        ],
    )(x_pad, W1, deg)


def _ln(v, g, b, eps=1e-5):
    mean = jnp.mean(v, axis=-1, keepdims=True)
    var = jnp.mean(jnp.square(v - mean), axis=-1, keepdims=True)
    return (v - mean) * lax.rsqrt(var + eps) * g + b


def _mid_body(s1_r, u1_r, dv_r, b1_r, g1_r, be1_r, g2_r, be2_r, w2_r,
              h_r, u2_r):
    dinv = dv_r[0, :]
    conv1 = dinv[:, None] * (s1_r[...] + u1_r[...]) + b1_r[...]
    h = jax.nn.relu(_ln(conv1, g1_r[...], be1_r[...]))
    t = jax.nn.relu(_ln(h, g2_r[...], be2_r[...]))
    h_r[...] = h
    u2_r[...] = jnp.dot(t * dinv[:, None], w2_r[...],
                        preferred_element_type=jnp.float32)


def _tc_mid(s1, u1, dinv, b1, g1, be1, g2, be2, W2):
    vec = lambda: pl.BlockSpec((1, D_HID), lambda i: (0, 0))
    return pl.pallas_call(
        _mid_body,
        grid=(GRID,),
        in_specs=[
            pl.BlockSpec((RB, D_HID), lambda i: (i, 0)),
            pl.BlockSpec((RB, D_HID), lambda i: (i, 0)),
            pl.BlockSpec((1, RB), lambda i: (0, i)),
            vec(), vec(), vec(), vec(), vec(),
            pl.BlockSpec((D_HID, D_HID), lambda i: (0, 0)),
        ],
        out_specs=[
            pl.BlockSpec((RB, D_HID), lambda i: (i, 0)),
            pl.BlockSpec((RB, D_HID), lambda i: (i, 0)),
        ],
        out_shape=[
            jax.ShapeDtypeStruct((NP, D_HID), jnp.float32),
            jax.ShapeDtypeStruct((NP, D_HID), jnp.float32),
        ],
    )(s1, u1, dinv, b1, g1, be1, g2, be2, W2)


def _fin_body(h_r, dv_r, s2_r, u2_r, b2_r, o_r):
    dinv = dv_r[0, :]
    o_r[...] = h_r[...] + dinv[:, None] * (s2_r[...] + u2_r[...]) + b2_r[...]


def _tc_fin(h, dinv, s2, u2, b2):
    return pl.pallas_call(
        _fin_body,
        grid=(GRID,),
        in_specs=[
            pl.BlockSpec((RB, D_HID), lambda i: (i, 0)),
            pl.BlockSpec((1, RB), lambda i: (0, i)),
            pl.BlockSpec((RB, D_HID), lambda i: (i, 0)),
            pl.BlockSpec((RB, D_HID), lambda i: (i, 0)),
            pl.BlockSpec((1, D_HID), lambda i: (0, 0)),
        ],
        out_specs=pl.BlockSpec((RB, D_HID), lambda i: (i, 0)),
        out_shape=jax.ShapeDtypeStruct((NP, D_HID), jnp.float32),
    )(h, dinv, s2, u2, b2)


def kernel(x, edge_index, W1, b1, g1, be1, W2, b2, g2, be2):
    src32 = edge_index[0].astype(jnp.int32)
    dst32 = edge_index[1].astype(jnp.int32)
    x_pad = jnp.pad(x, ((0, NP - N), (0, 0)))

    fsrc, fdst, cnt, deg = _sc_partition(src32, dst32)
    deg = deg.reshape(1, NP)
    u1, dinv = _tc_proj1(x_pad, W1, deg)
    s1 = _sc_segsum(u1, fsrc, fdst, cnt)
    h, u2 = _tc_mid(s1, u1, dinv,
                    b1.reshape(1, -1), g1.reshape(1, -1), be1.reshape(1, -1),
                    g2.reshape(1, -1), be2.reshape(1, -1), W2)
    s2 = _sc_segsum(u2, fsrc, fdst, cnt)
    out = _tc_fin(h, dinv, s2, u2, b2.reshape(1, -1))
    return out[:N]


# speedup vs baseline: 1.7696x; 1.3698x over previous
import jax
import jax.numpy as jnp
from jax import lax
from jax.experimental import pallas as pl
from jax.experimental.pallas import tpu as pltpu
from jax.experimental.pallas import tpu_sc as plsc

N = 10000
E = 320000
D_IN = 128
D_HID = 256
NP = 10240
NC = 2
NS = 16
NW = NC * NS
RPW = NP // NW
B_E = 4000
CAP = 11520
K = 32
ACC_ROWS = RPW + 1
DUMP = RPW
RB = 2560
GRID = NP // RB


def _sc_mesh():
    return plsc.VectorSubcoreMesh(
        core_axis_name="c", subcore_axis_name="s", num_cores=NC, num_subcores=NS
    )


_SC_PARAMS = pltpu.CompilerParams(needs_layout_passes=False)


def _part_body(src_hbm, dst_hbm, fsrc_hbm, fdst_hbm, cnt_hbm, deg_hbm,
               src_v0, dst_v0, src_v1, dst_v1, fsrc, fdst, deg_v, cnt_v,
               sem_s0, sem_d0, sem_s1, sem_d1):
    c = lax.axis_index("c")
    s = lax.axis_index("s")
    w = s * NC + c
    lo = w * RPW

    def zdeg(i, _):
        deg_v[pl.ds(i * 16, 16)] = jnp.zeros((16,), jnp.float32)
        return 0

    lax.fori_loop(0, RPW // 16, zdeg, 0)

    z16 = jnp.zeros((16,), jnp.int32)
    d16 = jnp.full((16,), DUMP, jnp.int32)

    def pre(i, _):
        fsrc[pl.ds(i * 16, 16)] = z16
        fdst[pl.ds(i * 16, 16)] = d16
        return 0

    lax.fori_loop(0, CAP // 16, pre, 0)

    ones = jnp.ones((16,), jnp.float32)
    NBLK = E // B_E

    def fetch(src_v, dst_v, sem_s, sem_d, blk):
        base_e = jnp.minimum(blk, NBLK - 1) * B_E
        pltpu.async_copy(src_hbm.at[pl.ds(base_e, B_E)], src_v, sem_s)
        pltpu.async_copy(dst_hbm.at[pl.ds(base_e, B_E)], dst_v, sem_d)

    def wait(src_v, dst_v, sem_s, sem_d):
        pltpu.make_async_copy(src_hbm.at[pl.ds(0, B_E)], src_v, sem_s).wait()
        pltpu.make_async_copy(dst_hbm.at[pl.ds(0, B_E)], dst_v, sem_d).wait()

    def filter_block(src_v, dst_v, pos):
        def filt(i, pos):
            for h in range(2):
                sl = src_v[pl.ds(i * 32 + h * 16, 16)]
                dl = dst_v[pl.ds(i * 32 + h * 16, 16)] - lo
                m = (dl >= 0) & (dl < RPW)
                dl_safe = jnp.clip(dl, 0, RPW - 1)
                pos_c = jnp.minimum(pos, CAP - 16)
                plsc.store_compressed(fsrc.at[pl.ds(pos_c, 16)], sl, mask=m)
                plsc.store_compressed(fdst.at[pl.ds(pos_c, 16)], dl_safe, mask=m)
                plsc.addupdate_scatter(deg_v, [dl_safe], ones, mask=m)
                pos = pos + jnp.sum(m.astype(jnp.int32))
            return pos

        return lax.fori_loop(0, B_E // 32, filt, pos)

    fetch(src_v0, dst_v0, sem_s0, sem_d0, 0)

    def pair(p, pos):
        fetch(src_v1, dst_v1, sem_s1, sem_d1, 2 * p + 1)
        wait(src_v0, dst_v0, sem_s0, sem_d0)
        pos = filter_block(src_v0, dst_v0, pos)
        fetch(src_v0, dst_v0, sem_s0, sem_d0, 2 * p + 2)
        wait(src_v1, dst_v1, sem_s1, sem_d1)
        return filter_block(src_v1, dst_v1, pos)

    pos = lax.fori_loop(0, NBLK // 2, pair, jnp.int32(0))
    wait(src_v0, dst_v0, sem_s0, sem_d0)

    cnt_v[pl.ds(0, 16)] = jnp.broadcast_to(jnp.minimum(pos, CAP), (16,))
    pltpu.sync_copy(fsrc, fsrc_hbm.at[w])
    pltpu.sync_copy(fdst, fdst_hbm.at[w])
    pltpu.sync_copy(cnt_v, cnt_hbm.at[w])
    pltpu.sync_copy(deg_v, deg_hbm.at[w])


def _sc_partition(src32, dst32):
    return pl.kernel(
        _part_body,
        out_type=[
            jax.ShapeDtypeStruct((NW, CAP), jnp.int32),
            jax.ShapeDtypeStruct((NW, CAP), jnp.int32),
            jax.ShapeDtypeStruct((NW, 16), jnp.int32),
            jax.ShapeDtypeStruct((NW, RPW), jnp.float32),
        ],
        mesh=_sc_mesh(),
        compiler_params=_SC_PARAMS,
        scratch_types=[
            pltpu.VMEM((B_E,), jnp.int32),
            pltpu.VMEM((B_E,), jnp.int32),
            pltpu.VMEM((B_E,), jnp.int32),
            pltpu.VMEM((B_E,), jnp.int32),
            pltpu.VMEM((CAP,), jnp.int32),
            pltpu.VMEM((CAP,), jnp.int32),
            pltpu.VMEM((RPW,), jnp.float32),
            pltpu.VMEM((16,), jnp.int32),
            pltpu.SemaphoreType.DMA,
            pltpu.SemaphoreType.DMA,
            pltpu.SemaphoreType.DMA,
            pltpu.SemaphoreType.DMA,
        ],
    )(src32, dst32)


CAPX = CAP + K


def _segsum_body(u_hbm, fsrc_hbm, fdst_hbm, cnt_hbm, out_hbm,
                 fsrc, fdst, rows0, rows1, cnt_v, acc,
                 sem0, sem1):
    c = lax.axis_index("c")
    s = lax.axis_index("s")
    w = s * NC + c

    def zacc(i, _):
        acc[i // 16, pl.ds((i % 16) * 16, 16)] = jnp.zeros((16,), jnp.float32)
        return 0

    lax.fori_loop(0, ACC_ROWS * 16, zacc, 0)

    pltpu.sync_copy(fsrc_hbm.at[w], fsrc.at[pl.ds(0, CAP)])
    pltpu.sync_copy(fdst_hbm.at[w], fdst.at[pl.ds(0, CAP)])
    pltpu.sync_copy(cnt_hbm.at[w], cnt_v)
    z16 = jnp.zeros((16,), jnp.int32)
    d16 = jnp.full((16,), DUMP, jnp.int32)
    for j in range(K // 16):
        fsrc[pl.ds(CAP + j * 16, 16)] = z16
        fdst[pl.ds(CAP + j * 16, 16)] = d16

    count = cnt_v[pl.ds(0, 16)][0]
    pairs = (count + (2 * K - 1)) // (2 * K)

    def add_batch(rows, t):
        base = t * K

        def per_group(g, _):
            dv = fdst[pl.ds(base + g * 16, 16)]
            for l in range(0, 16, 2):
                d0 = dv[l]
                d1 = dv[l + 1]
                i0 = g * 16 + l
                i1 = i0 + 1
                for j in range(D_HID // 16):
                    r0 = rows[i0, pl.ds(j * 16, 16)]
                    r1 = rows[i1, pl.ds(j * 16, 16)]
                    plsc.addupdate(acc.at[d0, pl.ds(j * 16, 16)], r0)
                    plsc.addupdate(acc.at[d1, pl.ds(j * 16, 16)], r1)
            return 0

        lax.fori_loop(0, K // 16, per_group, 0)

    def gather_at(t):
        return u_hbm.at[fsrc.at[pl.ds(t * K, K)]]

    pltpu.async_copy(gather_at(0), rows0, sem0)

    def pair(p, _):
        t0 = 2 * p
        pltpu.async_copy(gather_at(t0 + 1), rows1, sem1)
        pltpu.make_async_copy(gather_at(t0), rows0, sem0).wait()
        add_batch(rows0, t0)
        pltpu.async_copy(gather_at(t0 + 2), rows0, sem0)
        pltpu.make_async_copy(gather_at(t0 + 1), rows1, sem1).wait()
        add_batch(rows1, t0 + 1)
        return 0

    lax.fori_loop(0, pairs, pair, 0)
    pltpu.make_async_copy(gather_at(0), rows0, sem0).wait()

    pltpu.sync_copy(acc.at[pl.ds(0, RPW)], out_hbm.at[pl.ds(w * RPW, RPW)])


def _sc_segsum(u, fsrc, fdst, cnt):
    return pl.kernel(
        _segsum_body,
        out_type=jax.ShapeDtypeStruct((NP, D_HID), jnp.float32),
        mesh=_sc_mesh(),
        compiler_params=_SC_PARAMS,
        scratch_types=[
            pltpu.VMEM((CAPX,), jnp.int32),
            pltpu.VMEM((CAPX,), jnp.int32),
            pltpu.VMEM((K, D_HID), jnp.float32),
            pltpu.VMEM((K, D_HID), jnp.float32),
            pltpu.VMEM((16,), jnp.int32),
            pltpu.VMEM((ACC_ROWS, D_HID), jnp.float32),
            pltpu.SemaphoreType.DMA,
            pltpu.SemaphoreType.DMA,
        ],
    )(u, fsrc, fdst, cnt)


def _proj1_body(x_r, w_r, dg_r, u1_r, dv_r):
    deg = dg_r[0, :] + 1.0
    dinv = lax.rsqrt(jnp.maximum(deg, 1e-12))
    dv_r[...] = dinv[None, :]
    u1_r[...] = jnp.dot(x_r[...] * dinv[:, None], w_r[...],
                        preferred_element_type=jnp.float32)


def _tc_proj1(x_pad, W1, deg):
    return pl.pallas_call(
        _proj1_body,
        grid=(GRID,),
        in_specs=[
            pl.BlockSpec((RB, D_IN), lambda i: (i, 0)),
            pl.BlockSpec((D_IN, D_HID), lambda i: (0, 0)),
            pl.BlockSpec((1, RB), lambda i: (0, i)),
        ],
        out_specs=[
            pl.BlockSpec((RB, D_HID), lambda i: (i, 0)),
            pl.BlockSpec((1, RB), lambda i: (0, i)),
        ],
        out_shape=[
            jax.ShapeDtypeStruct((NP, D_HID), jnp.float32),
            jax.ShapeDtypeStruct((1, NP), jnp.float32),
        ],
    )(x_pad, W1, deg)


def _ln(v, g, b, eps=1e-5):
    mean = jnp.mean(v, axis=-1, keepdims=True)
    var = jnp.mean(jnp.square(v - mean), axis=-1, keepdims=True)
    return (v - mean) * lax.rsqrt(var + eps) * g + b


def _mid_body(s1_r, u1_r, dv_r, b1_r, g1_r, be1_r, g2_r, be2_r, w2_r,
              h_r, u2_r):
    dinv = dv_r[0, :]
    conv1 = dinv[:, None] * (s1_r[...] + u1_r[...]) + b1_r[...]
    h = jax.nn.relu(_ln(conv1, g1_r[...], be1_r[...]))
    t = jax.nn.relu(_ln(h, g2_r[...], be2_r[...]))
    h_r[...] = h
    u2_r[...] = jnp.dot(t * dinv[:, None], w2_r[...],
                        preferred_element_type=jnp.float32)


def _tc_mid(s1, u1, dinv, b1, g1, be1, g2, be2, W2):
    vec = lambda: pl.BlockSpec((1, D_HID), lambda i: (0, 0))
    return pl.pallas_call(
        _mid_body,
        grid=(GRID,),
        in_specs=[
            pl.BlockSpec((RB, D_HID), lambda i: (i, 0)),
            pl.BlockSpec((RB, D_HID), lambda i: (i, 0)),
            pl.BlockSpec((1, RB), lambda i: (0, i)),
            vec(), vec(), vec(), vec(), vec(),
            pl.BlockSpec((D_HID, D_HID), lambda i: (0, 0)),
        ],
        out_specs=[
            pl.BlockSpec((RB, D_HID), lambda i: (i, 0)),
            pl.BlockSpec((RB, D_HID), lambda i: (i, 0)),
        ],
        out_shape=[
            jax.ShapeDtypeStruct((NP, D_HID), jnp.float32),
            jax.ShapeDtypeStruct((NP, D_HID), jnp.float32),
        ],
    )(s1, u1, dinv, b1, g1, be1, g2, be2, W2)


def _fin_body(h_r, dv_r, s2_r, u2_r, b2_r, o_r):
    dinv = dv_r[0, :]
    o_r[...] = h_r[...] + dinv[:, None] * (s2_r[...] + u2_r[...]) + b2_r[...]


def _tc_fin(h, dinv, s2, u2, b2):
    return pl.pallas_call(
        _fin_body,
        grid=(GRID,),
        in_specs=[
            pl.BlockSpec((RB, D_HID), lambda i: (i, 0)),
            pl.BlockSpec((1, RB), lambda i: (0, i)),
            pl.BlockSpec((RB, D_HID), lambda i: (i, 0)),
            pl.BlockSpec((RB, D_HID), lambda i: (i, 0)),
            pl.BlockSpec((1, D_HID), lambda i: (0, 0)),
        ],
        out_specs=pl.BlockSpec((RB, D_HID), lambda i: (i, 0)),
        out_shape=jax.ShapeDtypeStruct((NP, D_HID), jnp.float32),
    )(h, dinv, s2, u2, b2)


def kernel(x, edge_index, W1, b1, g1, be1, W2, b2, g2, be2):
    src32 = edge_index[0].astype(jnp.int32)
    dst32 = edge_index[1].astype(jnp.int32)
    x_pad = jnp.pad(x, ((0, NP - N), (0, 0)))

    fsrc, fdst, cnt, deg = _sc_partition(src32, dst32)
    deg = deg.reshape(1, NP)
    u1, dinv = _tc_proj1(x_pad, W1, deg)
    s1 = _sc_segsum(u1, fsrc, fdst, cnt)
    h, u2 = _tc_mid(s1, u1, dinv,
                    b1.reshape(1, -1), g1.reshape(1, -1), be1.reshape(1, -1),
                    g2.reshape(1, -1), be2.reshape(1, -1), W2)
    s2 = _sc_segsum(u2, fsrc, fdst, cnt)
    out = _tc_fin(h, dinv, s2, u2, b2.reshape(1, -1))
    return out[:N]


# speedup vs baseline: 2.0204x; 1.1418x over previous
import jax
import jax.numpy as jnp
from jax import lax
from jax.experimental import pallas as pl
from jax.experimental.pallas import tpu as pltpu
from jax.experimental.pallas import tpu_sc as plsc

N = 10000
E = 320000
D_IN = 128
D_HID = 256
NP = 10240
NC = 2
NS = 16
NW = NC * NS
RPW = NP // NW
B_E = 4000
CAP = 11520
K = 32
ACC_ROWS = RPW + 1
DUMP = RPW
RB = 2560
GRID = NP // RB


def _sc_mesh():
    return plsc.VectorSubcoreMesh(
        core_axis_name="c", subcore_axis_name="s", num_cores=NC, num_subcores=NS
    )


_SC_PARAMS = pltpu.CompilerParams(needs_layout_passes=False)


def _part_body(src_hbm, dst_hbm, fsrc_hbm, fdst_hbm, cnt_hbm, deg_hbm,
               src_v0, dst_v0, src_v1, dst_v1, fsrc, fdst, deg_v, cnt_v,
               sem_s0, sem_d0, sem_s1, sem_d1):
    c = lax.axis_index("c")
    s = lax.axis_index("s")
    w = s * NC + c
    lo = w * RPW

    def zdeg(i, _):
        deg_v[pl.ds(i * 16, 16)] = jnp.zeros((16,), jnp.float32)
        return 0

    lax.fori_loop(0, RPW // 16, zdeg, 0)

    z16 = jnp.zeros((16,), jnp.int32)
    d16 = jnp.full((16,), DUMP, jnp.int32)

    def pre(i, _):
        fsrc[pl.ds(i * 16, 16)] = z16
        fdst[pl.ds(i * 16, 16)] = d16
        return 0

    lax.fori_loop(0, CAP // 16, pre, 0)

    ones = jnp.ones((16,), jnp.float32)
    NBLK = E // B_E

    def fetch(src_v, dst_v, sem_s, sem_d, blk):
        base_e = jnp.minimum(blk, NBLK - 1) * B_E
        pltpu.async_copy(src_hbm.at[pl.ds(base_e, B_E)], src_v, sem_s)
        pltpu.async_copy(dst_hbm.at[pl.ds(base_e, B_E)], dst_v, sem_d)

    def wait(src_v, dst_v, sem_s, sem_d):
        pltpu.make_async_copy(src_hbm.at[pl.ds(0, B_E)], src_v, sem_s).wait()
        pltpu.make_async_copy(dst_hbm.at[pl.ds(0, B_E)], dst_v, sem_d).wait()

    def filter_block(src_v, dst_v, pos):
        def filt(i, pos):
            for h in range(2):
                sl = src_v[pl.ds(i * 32 + h * 16, 16)]
                dl = dst_v[pl.ds(i * 32 + h * 16, 16)] - lo
                m = (dl >= 0) & (dl < RPW)
                dl_safe = jnp.clip(dl, 0, RPW - 1)
                pos_c = jnp.minimum(pos, CAP - 16)
                plsc.store_compressed(fsrc.at[pl.ds(pos_c, 16)], sl, mask=m)
                plsc.store_compressed(fdst.at[pl.ds(pos_c, 16)], dl_safe, mask=m)
                plsc.addupdate_scatter(deg_v, [dl_safe], ones, mask=m)
                pos = pos + jnp.sum(m.astype(jnp.int32))
            return pos

        return lax.fori_loop(0, B_E // 32, filt, pos)

    fetch(src_v0, dst_v0, sem_s0, sem_d0, 0)

    def pair(p, pos):
        fetch(src_v1, dst_v1, sem_s1, sem_d1, 2 * p + 1)
        wait(src_v0, dst_v0, sem_s0, sem_d0)
        pos = filter_block(src_v0, dst_v0, pos)
        fetch(src_v0, dst_v0, sem_s0, sem_d0, 2 * p + 2)
        wait(src_v1, dst_v1, sem_s1, sem_d1)
        return filter_block(src_v1, dst_v1, pos)

    pos = lax.fori_loop(0, NBLK // 2, pair, jnp.int32(0))
    wait(src_v0, dst_v0, sem_s0, sem_d0)

    cnt_v[pl.ds(0, 16)] = jnp.broadcast_to(jnp.minimum(pos, CAP), (16,))
    pltpu.sync_copy(fsrc, fsrc_hbm.at[w])
    pltpu.sync_copy(fdst, fdst_hbm.at[w])
    pltpu.sync_copy(cnt_v, cnt_hbm.at[w])
    pltpu.sync_copy(deg_v, deg_hbm.at[w])


def _sc_partition(src32, dst32):
    return pl.kernel(
        _part_body,
        out_type=[
            jax.ShapeDtypeStruct((NW, CAP), jnp.int32),
            jax.ShapeDtypeStruct((NW, CAP), jnp.int32),
            jax.ShapeDtypeStruct((NW, 16), jnp.int32),
            jax.ShapeDtypeStruct((NW, RPW), jnp.float32),
        ],
        mesh=_sc_mesh(),
        compiler_params=_SC_PARAMS,
        scratch_types=[
            pltpu.VMEM((B_E,), jnp.int32),
            pltpu.VMEM((B_E,), jnp.int32),
            pltpu.VMEM((B_E,), jnp.int32),
            pltpu.VMEM((B_E,), jnp.int32),
            pltpu.VMEM((CAP,), jnp.int32),
            pltpu.VMEM((CAP,), jnp.int32),
            pltpu.VMEM((RPW,), jnp.float32),
            pltpu.VMEM((16,), jnp.int32),
            pltpu.SemaphoreType.DMA,
            pltpu.SemaphoreType.DMA,
            pltpu.SemaphoreType.DMA,
            pltpu.SemaphoreType.DMA,
        ],
    )(src32, dst32)


CAPX = CAP + K


def _segsum_body(u_hbm, fsrc_hbm, fdst_hbm, cnt_hbm, out_hbm,
                 fsrc, fdst, rows0, rows1, cnt_v, acc,
                 sem0, sem1):
    c = lax.axis_index("c")
    s = lax.axis_index("s")
    w = s * NC + c

    def zacc(i, _):
        acc[i // 16, pl.ds((i % 16) * 16, 16)] = jnp.zeros((16,), jnp.float32)
        return 0

    lax.fori_loop(0, ACC_ROWS * 16, zacc, 0)

    pltpu.sync_copy(fsrc_hbm.at[w], fsrc.at[pl.ds(0, CAP)])
    pltpu.sync_copy(fdst_hbm.at[w], fdst.at[pl.ds(0, CAP)])
    pltpu.sync_copy(cnt_hbm.at[w], cnt_v)
    z16 = jnp.zeros((16,), jnp.int32)
    d16 = jnp.full((16,), DUMP, jnp.int32)
    for j in range(K // 16):
        fsrc[pl.ds(CAP + j * 16, 16)] = z16
        fdst[pl.ds(CAP + j * 16, 16)] = d16

    count = cnt_v[pl.ds(0, 16)][0]
    pairs = (count + (2 * K - 1)) // (2 * K)

    def add_batch(rows, t):
        base = t * K

        def per_group(g, _):
            dv = fdst[pl.ds(base + g * 16, 16)]
            for l in range(0, 16, 4):
                dd = [dv[l + q] for q in range(4)]
                ii = [g * 16 + l + q for q in range(4)]
                for j in range(D_HID // 16):
                    rr = [rows[i, pl.ds(j * 16, 16)] for i in ii]
                    for q in range(4):
                        plsc.addupdate(acc.at[dd[q], pl.ds(j * 16, 16)], rr[q])
            return 0

        lax.fori_loop(0, K // 16, per_group, 0)

    def gather_at(t):
        return u_hbm.at[fsrc.at[pl.ds(t * K, K)]]

    pltpu.async_copy(gather_at(0), rows0, sem0)

    def pair(p, _):
        t0 = 2 * p
        pltpu.async_copy(gather_at(t0 + 1), rows1, sem1)
        pltpu.make_async_copy(gather_at(t0), rows0, sem0).wait()
        add_batch(rows0, t0)
        pltpu.async_copy(gather_at(t0 + 2), rows0, sem0)
        pltpu.make_async_copy(gather_at(t0 + 1), rows1, sem1).wait()
        add_batch(rows1, t0 + 1)
        return 0

    lax.fori_loop(0, pairs, pair, 0)
    pltpu.make_async_copy(gather_at(0), rows0, sem0).wait()

    pltpu.sync_copy(acc.at[pl.ds(0, RPW)], out_hbm.at[pl.ds(w * RPW, RPW)])


def _sc_segsum(u, fsrc, fdst, cnt):
    return pl.kernel(
        _segsum_body,
        out_type=jax.ShapeDtypeStruct((NP, D_HID), jnp.float32),
        mesh=_sc_mesh(),
        compiler_params=_SC_PARAMS,
        scratch_types=[
            pltpu.VMEM((CAPX,), jnp.int32),
            pltpu.VMEM((CAPX,), jnp.int32),
            pltpu.VMEM((K, D_HID), jnp.float32),
            pltpu.VMEM((K, D_HID), jnp.float32),
            pltpu.VMEM((16,), jnp.int32),
            pltpu.VMEM((ACC_ROWS, D_HID), jnp.float32),
            pltpu.SemaphoreType.DMA,
            pltpu.SemaphoreType.DMA,
        ],
    )(u, fsrc, fdst, cnt)


def _proj1_body(x_r, w_r, dg_r, u1_r, dv_r):
    deg = dg_r[0, :] + 1.0
    dinv = lax.rsqrt(jnp.maximum(deg, 1e-12))
    dv_r[...] = dinv[None, :]
    u1_r[...] = jnp.dot(x_r[...] * dinv[:, None], w_r[...],
                        preferred_element_type=jnp.float32)


def _tc_proj1(x_pad, W1, deg):
    return pl.pallas_call(
        _proj1_body,
        grid=(GRID,),
        in_specs=[
            pl.BlockSpec((RB, D_IN), lambda i: (i, 0)),
            pl.BlockSpec((D_IN, D_HID), lambda i: (0, 0)),
            pl.BlockSpec((1, RB), lambda i: (0, i)),
        ],
        out_specs=[
            pl.BlockSpec((RB, D_HID), lambda i: (i, 0)),
            pl.BlockSpec((1, RB), lambda i: (0, i)),
        ],
        out_shape=[
            jax.ShapeDtypeStruct((NP, D_HID), jnp.float32),
            jax.ShapeDtypeStruct((1, NP), jnp.float32),
        ],
    )(x_pad, W1, deg)


def _ln(v, g, b, eps=1e-5):
    mean = jnp.mean(v, axis=-1, keepdims=True)
    var = jnp.mean(jnp.square(v - mean), axis=-1, keepdims=True)
    return (v - mean) * lax.rsqrt(var + eps) * g + b


def _mid_body(s1_r, u1_r, dv_r, b1_r, g1_r, be1_r, g2_r, be2_r, w2_r,
              h_r, u2_r):
    dinv = dv_r[0, :]
    conv1 = dinv[:, None] * (s1_r[...] + u1_r[...]) + b1_r[...]
    h = jax.nn.relu(_ln(conv1, g1_r[...], be1_r[...]))
    t = jax.nn.relu(_ln(h, g2_r[...], be2_r[...]))
    h_r[...] = h
    u2_r[...] = jnp.dot(t * dinv[:, None], w2_r[...],
                        preferred_element_type=jnp.float32)


def _tc_mid(s1, u1, dinv, b1, g1, be1, g2, be2, W2):
    vec = lambda: pl.BlockSpec((1, D_HID), lambda i: (0, 0))
    return pl.pallas_call(
        _mid_body,
        grid=(GRID,),
        in_specs=[
            pl.BlockSpec((RB, D_HID), lambda i: (i, 0)),
            pl.BlockSpec((RB, D_HID), lambda i: (i, 0)),
            pl.BlockSpec((1, RB), lambda i: (0, i)),
            vec(), vec(), vec(), vec(), vec(),
            pl.BlockSpec((D_HID, D_HID), lambda i: (0, 0)),
        ],
        out_specs=[
            pl.BlockSpec((RB, D_HID), lambda i: (i, 0)),
            pl.BlockSpec((RB, D_HID), lambda i: (i, 0)),
        ],
        out_shape=[
            jax.ShapeDtypeStruct((NP, D_HID), jnp.float32),
            jax.ShapeDtypeStruct((NP, D_HID), jnp.float32),
        ],
    )(s1, u1, dinv, b1, g1, be1, g2, be2, W2)


def _fin_body(h_r, dv_r, s2_r, u2_r, b2_r, o_r):
    dinv = dv_r[0, :]
    o_r[...] = h_r[...] + dinv[:, None] * (s2_r[...] + u2_r[...]) + b2_r[...]


def _tc_fin(h, dinv, s2, u2, b2):
    return pl.pallas_call(
        _fin_body,
        grid=(GRID,),
        in_specs=[
            pl.BlockSpec((RB, D_HID), lambda i: (i, 0)),
            pl.BlockSpec((1, RB), lambda i: (0, i)),
            pl.BlockSpec((RB, D_HID), lambda i: (i, 0)),
            pl.BlockSpec((RB, D_HID), lambda i: (i, 0)),
            pl.BlockSpec((1, D_HID), lambda i: (0, 0)),
        ],
        out_specs=pl.BlockSpec((RB, D_HID), lambda i: (i, 0)),
        out_shape=jax.ShapeDtypeStruct((NP, D_HID), jnp.float32),
    )(h, dinv, s2, u2, b2)


def kernel(x, edge_index, W1, b1, g1, be1, W2, b2, g2, be2):
    src32 = edge_index[0].astype(jnp.int32)
    dst32 = edge_index[1].astype(jnp.int32)
    x_pad = jnp.pad(x, ((0, NP - N), (0, 0)))

    fsrc, fdst, cnt, deg = _sc_partition(src32, dst32)
    deg = deg.reshape(1, NP)
    u1, dinv = _tc_proj1(x_pad, W1, deg)
    s1 = _sc_segsum(u1, fsrc, fdst, cnt)
    h, u2 = _tc_mid(s1, u1, dinv,
                    b1.reshape(1, -1), g1.reshape(1, -1), be1.reshape(1, -1),
                    g2.reshape(1, -1), be2.reshape(1, -1), W2)
    s2 = _sc_segsum(u2, fsrc, fdst, cnt)
    out = _tc_fin(h, dinv, s2, u2, b2.reshape(1, -1))
    return out[:N]


# speedup vs baseline: 2.0607x; 1.0199x over previous
import jax
import jax.numpy as jnp
from jax import lax
from jax.experimental import pallas as pl
from jax.experimental.pallas import tpu as pltpu
from jax.experimental.pallas import tpu_sc as plsc

N = 10000
E = 320000
D_IN = 128
D_HID = 256
NP = 10240
NC = 2
NS = 16
NW = NC * NS
RPW = NP // NW
B_E = 4000
CAP = 11520
K = 32
ACC_ROWS = RPW + 1
DUMP = RPW
RB = 2560
GRID = NP // RB


def _sc_mesh():
    return plsc.VectorSubcoreMesh(
        core_axis_name="c", subcore_axis_name="s", num_cores=NC, num_subcores=NS
    )


_SC_PARAMS = pltpu.CompilerParams(needs_layout_passes=False)


def _part_body(src_hbm, dst_hbm, fsrc_hbm, fdst_hbm, cnt_hbm, deg_hbm,
               src_v0, dst_v0, src_v1, dst_v1, fsrc, fdst, deg_v, cnt_v,
               sem_s0, sem_d0, sem_s1, sem_d1):
    c = lax.axis_index("c")
    s = lax.axis_index("s")
    w = s * NC + c
    lo = w * RPW

    def zdeg(i, _):
        deg_v[pl.ds(i * 16, 16)] = jnp.zeros((16,), jnp.float32)
        return 0

    lax.fori_loop(0, RPW // 16, zdeg, 0)

    z16 = jnp.zeros((16,), jnp.int32)
    d16 = jnp.full((16,), DUMP, jnp.int32)

    def pre(i, _):
        fsrc[pl.ds(i * 16, 16)] = z16
        fdst[pl.ds(i * 16, 16)] = d16
        return 0

    lax.fori_loop(0, CAP // 16, pre, 0)

    ones = jnp.ones((16,), jnp.float32)
    NBLK = E // B_E

    def fetch(src_v, dst_v, sem_s, sem_d, blk):
        base_e = jnp.minimum(blk, NBLK - 1) * B_E
        pltpu.async_copy(src_hbm.at[pl.ds(base_e, B_E)], src_v, sem_s)
        pltpu.async_copy(dst_hbm.at[pl.ds(base_e, B_E)], dst_v, sem_d)

    def wait(src_v, dst_v, sem_s, sem_d):
        pltpu.make_async_copy(src_hbm.at[pl.ds(0, B_E)], src_v, sem_s).wait()
        pltpu.make_async_copy(dst_hbm.at[pl.ds(0, B_E)], dst_v, sem_d).wait()

    def filter_block(src_v, dst_v, pos):
        def filt(i, pos):
            for h in range(2):
                sl = src_v[pl.ds(i * 32 + h * 16, 16)]
                dl = dst_v[pl.ds(i * 32 + h * 16, 16)] - lo
                m = (dl >= 0) & (dl < RPW)
                dl_safe = jnp.clip(dl, 0, RPW - 1)
                pos_c = jnp.minimum(pos, CAP - 16)
                plsc.store_compressed(fsrc.at[pl.ds(pos_c, 16)], sl, mask=m)
                plsc.store_compressed(fdst.at[pl.ds(pos_c, 16)], dl_safe, mask=m)
                plsc.addupdate_scatter(deg_v, [dl_safe], ones, mask=m)
                pos = pos + jnp.sum(m.astype(jnp.int32))
            return pos

        return lax.fori_loop(0, B_E // 32, filt, pos)

    fetch(src_v0, dst_v0, sem_s0, sem_d0, 0)

    def pair(p, pos):
        fetch(src_v1, dst_v1, sem_s1, sem_d1, 2 * p + 1)
        wait(src_v0, dst_v0, sem_s0, sem_d0)
        pos = filter_block(src_v0, dst_v0, pos)
        fetch(src_v0, dst_v0, sem_s0, sem_d0, 2 * p + 2)
        wait(src_v1, dst_v1, sem_s1, sem_d1)
        return filter_block(src_v1, dst_v1, pos)

    pos = lax.fori_loop(0, NBLK // 2, pair, jnp.int32(0))
    wait(src_v0, dst_v0, sem_s0, sem_d0)

    cnt_v[pl.ds(0, 16)] = jnp.broadcast_to(jnp.minimum(pos, CAP), (16,))
    pltpu.sync_copy(fsrc, fsrc_hbm.at[w])
    pltpu.sync_copy(fdst, fdst_hbm.at[w])
    pltpu.sync_copy(cnt_v, cnt_hbm.at[w])
    pltpu.sync_copy(deg_v, deg_hbm.at[w])


def _sc_partition(src32, dst32):
    return pl.kernel(
        _part_body,
        out_type=[
            jax.ShapeDtypeStruct((NW, CAP), jnp.int32),
            jax.ShapeDtypeStruct((NW, CAP), jnp.int32),
            jax.ShapeDtypeStruct((NW, 16), jnp.int32),
            jax.ShapeDtypeStruct((NW, RPW), jnp.float32),
        ],
        mesh=_sc_mesh(),
        compiler_params=_SC_PARAMS,
        scratch_types=[
            pltpu.VMEM((B_E,), jnp.int32),
            pltpu.VMEM((B_E,), jnp.int32),
            pltpu.VMEM((B_E,), jnp.int32),
            pltpu.VMEM((B_E,), jnp.int32),
            pltpu.VMEM((CAP,), jnp.int32),
            pltpu.VMEM((CAP,), jnp.int32),
            pltpu.VMEM((RPW,), jnp.float32),
            pltpu.VMEM((16,), jnp.int32),
            pltpu.SemaphoreType.DMA,
            pltpu.SemaphoreType.DMA,
            pltpu.SemaphoreType.DMA,
            pltpu.SemaphoreType.DMA,
        ],
    )(src32, dst32)


CAPX = CAP + K


def _segsum_body(u_hbm, fsrc_hbm, fdst_hbm, cnt_hbm, out_hbm,
                 fsrc, fdst, rows0, rows1, cnt_v, acc,
                 sem0, sem1):
    c = lax.axis_index("c")
    s = lax.axis_index("s")
    w = s * NC + c

    def zacc(i, _):
        acc[i // 16, pl.ds((i % 16) * 16, 16)] = jnp.zeros((16,), jnp.float32)
        return 0

    lax.fori_loop(0, ACC_ROWS * 16, zacc, 0)

    pltpu.sync_copy(fsrc_hbm.at[w], fsrc.at[pl.ds(0, CAP)])
    pltpu.sync_copy(fdst_hbm.at[w], fdst.at[pl.ds(0, CAP)])
    pltpu.sync_copy(cnt_hbm.at[w], cnt_v)
    z16 = jnp.zeros((16,), jnp.int32)
    d16 = jnp.full((16,), DUMP, jnp.int32)
    for j in range(K // 16):
        fsrc[pl.ds(CAP + j * 16, 16)] = z16
        fdst[pl.ds(CAP + j * 16, 16)] = d16

    count = cnt_v[pl.ds(0, 16)][0]
    pairs = (count + (2 * K - 1)) // (2 * K)

    def add_batch(rows, t):
        base = t * K

        def per_group(g, _):
            dv = fdst[pl.ds(base + g * 16, 16)]
            for l in range(0, 16, 8):
                dd = [dv[l + q] for q in range(8)]
                ii = [g * 16 + l + q for q in range(8)]
                for j in range(D_HID // 16):
                    rr = [rows[i, pl.ds(j * 16, 16)] for i in ii]
                    for q in range(8):
                        plsc.addupdate(acc.at[dd[q], pl.ds(j * 16, 16)], rr[q])
            return 0

        lax.fori_loop(0, K // 16, per_group, 0)

    def gather_at(t):
        return u_hbm.at[fsrc.at[pl.ds(t * K, K)]]

    pltpu.async_copy(gather_at(0), rows0, sem0)

    def pair(p, _):
        t0 = 2 * p
        pltpu.async_copy(gather_at(t0 + 1), rows1, sem1)
        pltpu.make_async_copy(gather_at(t0), rows0, sem0).wait()
        add_batch(rows0, t0)
        pltpu.async_copy(gather_at(t0 + 2), rows0, sem0)
        pltpu.make_async_copy(gather_at(t0 + 1), rows1, sem1).wait()
        add_batch(rows1, t0 + 1)
        return 0

    lax.fori_loop(0, pairs, pair, 0)
    pltpu.make_async_copy(gather_at(0), rows0, sem0).wait()

    pltpu.sync_copy(acc.at[pl.ds(0, RPW)], out_hbm.at[pl.ds(w * RPW, RPW)])


def _sc_segsum(u, fsrc, fdst, cnt):
    return pl.kernel(
        _segsum_body,
        out_type=jax.ShapeDtypeStruct((NP, D_HID), jnp.float32),
        mesh=_sc_mesh(),
        compiler_params=_SC_PARAMS,
        scratch_types=[
            pltpu.VMEM((CAPX,), jnp.int32),
            pltpu.VMEM((CAPX,), jnp.int32),
            pltpu.VMEM((K, D_HID), jnp.float32),
            pltpu.VMEM((K, D_HID), jnp.float32),
            pltpu.VMEM((16,), jnp.int32),
            pltpu.VMEM((ACC_ROWS, D_HID), jnp.float32),
            pltpu.SemaphoreType.DMA,
            pltpu.SemaphoreType.DMA,
        ],
    )(u, fsrc, fdst, cnt)


def _proj1_body(x_r, w_r, dg_r, u1_r, dv_r):
    deg = dg_r[0, :] + 1.0
    dinv = lax.rsqrt(jnp.maximum(deg, 1e-12))
    dv_r[...] = dinv[None, :]
    u1_r[...] = jnp.dot(x_r[...] * dinv[:, None], w_r[...],
                        preferred_element_type=jnp.float32)


def _tc_proj1(x_pad, W1, deg):
    return pl.pallas_call(
        _proj1_body,
        grid=(GRID,),
        in_specs=[
            pl.BlockSpec((RB, D_IN), lambda i: (i, 0)),
            pl.BlockSpec((D_IN, D_HID), lambda i: (0, 0)),
            pl.BlockSpec((1, RB), lambda i: (0, i)),
        ],
        out_specs=[
            pl.BlockSpec((RB, D_HID), lambda i: (i, 0)),
            pl.BlockSpec((1, RB), lambda i: (0, i)),
        ],
        out_shape=[
            jax.ShapeDtypeStruct((NP, D_HID), jnp.float32),
            jax.ShapeDtypeStruct((1, NP), jnp.float32),
        ],
    )(x_pad, W1, deg)


def _ln(v, g, b, eps=1e-5):
    mean = jnp.mean(v, axis=-1, keepdims=True)
    var = jnp.mean(jnp.square(v - mean), axis=-1, keepdims=True)
    return (v - mean) * lax.rsqrt(var + eps) * g + b


def _mid_body(s1_r, u1_r, dv_r, b1_r, g1_r, be1_r, g2_r, be2_r, w2_r,
              h_r, u2_r):
    dinv = dv_r[0, :]
    conv1 = dinv[:, None] * (s1_r[...] + u1_r[...]) + b1_r[...]
    h = jax.nn.relu(_ln(conv1, g1_r[...], be1_r[...]))
    t = jax.nn.relu(_ln(h, g2_r[...], be2_r[...]))
    h_r[...] = h
    u2_r[...] = jnp.dot(t * dinv[:, None], w2_r[...],
                        preferred_element_type=jnp.float32)


def _tc_mid(s1, u1, dinv, b1, g1, be1, g2, be2, W2):
    vec = lambda: pl.BlockSpec((1, D_HID), lambda i: (0, 0))
    return pl.pallas_call(
        _mid_body,
        grid=(GRID,),
        in_specs=[
            pl.BlockSpec((RB, D_HID), lambda i: (i, 0)),
            pl.BlockSpec((RB, D_HID), lambda i: (i, 0)),
            pl.BlockSpec((1, RB), lambda i: (0, i)),
            vec(), vec(), vec(), vec(), vec(),
            pl.BlockSpec((D_HID, D_HID), lambda i: (0, 0)),
        ],
        out_specs=[
            pl.BlockSpec((RB, D_HID), lambda i: (i, 0)),
            pl.BlockSpec((RB, D_HID), lambda i: (i, 0)),
        ],
        out_shape=[
            jax.ShapeDtypeStruct((NP, D_HID), jnp.float32),
            jax.ShapeDtypeStruct((NP, D_HID), jnp.float32),
        ],
    )(s1, u1, dinv, b1, g1, be1, g2, be2, W2)


def _fin_body(h_r, dv_r, s2_r, u2_r, b2_r, o_r):
    dinv = dv_r[0, :]
    o_r[...] = h_r[...] + dinv[:, None] * (s2_r[...] + u2_r[...]) + b2_r[...]


def _tc_fin(h, dinv, s2, u2, b2):
    return pl.pallas_call(
        _fin_body,
        grid=(GRID,),
        in_specs=[
            pl.BlockSpec((RB, D_HID), lambda i: (i, 0)),
            pl.BlockSpec((1, RB), lambda i: (0, i)),
            pl.BlockSpec((RB, D_HID), lambda i: (i, 0)),
            pl.BlockSpec((RB, D_HID), lambda i: (i, 0)),
            pl.BlockSpec((1, D_HID), lambda i: (0, 0)),
        ],
        out_specs=pl.BlockSpec((RB, D_HID), lambda i: (i, 0)),
        out_shape=jax.ShapeDtypeStruct((NP, D_HID), jnp.float32),
    )(h, dinv, s2, u2, b2)


def kernel(x, edge_index, W1, b1, g1, be1, W2, b2, g2, be2):
    src32 = edge_index[0].astype(jnp.int32)
    dst32 = edge_index[1].astype(jnp.int32)
    x_pad = jnp.pad(x, ((0, NP - N), (0, 0)))

    fsrc, fdst, cnt, deg = _sc_partition(src32, dst32)
    deg = deg.reshape(1, NP)
    u1, dinv = _tc_proj1(x_pad, W1, deg)
    s1 = _sc_segsum(u1, fsrc, fdst, cnt)
    h, u2 = _tc_mid(s1, u1, dinv,
                    b1.reshape(1, -1), g1.reshape(1, -1), be1.reshape(1, -1),
                    g2.reshape(1, -1), be2.reshape(1, -1), W2)
    s2 = _sc_segsum(u2, fsrc, fdst, cnt)
    out = _tc_fin(h, dinv, s2, u2, b2.reshape(1, -1))
    return out[:N]
